# baseline (speedup 1.0000x reference)
import jax
import jax.numpy as jnp
from jax.experimental import pallas as pl
from jax.experimental.pallas import tpu as pltpu

NUM_STEPS = 64
NUM_AGENTS = 500000
QUARANTINE_DAYS = 10.0
_SUB = 8
_W = NUM_AGENTS // _SUB


def _threefry2x32(k0, k1, x1_in):
    ks2 = k0 ^ k1 ^ jnp.uint32(0x1BD11BDA)
    ks = (k0, k1, ks2)
    x0 = jnp.zeros_like(x1_in) + k0
    x1 = x1_in + k1
    rotations = ((13, 15, 26, 6), (17, 29, 16, 24))
    for i in range(5):
        for r in rotations[i % 2]:
            x0 = x0 + x1
            x1 = (x1 << jnp.uint32(r)) | (x1 >> jnp.uint32(32 - r))
            x1 = x1 ^ x0
        x0 = x0 + ks[(i + 1) % 3]
        x1 = x1 + ks[(i + 2) % 3] + jnp.uint32(i + 1)
    return x0 ^ x1


def _bits_to_unit(bits):
    f = jax.lax.bitcast_convert_type(
        (bits >> jnp.uint32(9)) | jnp.uint32(0x3F800000), jnp.float32
    ) - jnp.float32(1.0)
    minv = jnp.float32(1e-6)
    maxv = jnp.float32(1.0 - 1e-6)
    return jnp.maximum(minv, f * (maxv - minv) + minv)


def _body(kd_ref, probs_ref, t_ref, iq_ref, qsd_ref, out_ref,
          vrow, vqsd, vout, sem_big, sem_row, sem_qsd, sem_wr):
    tt = t_ref[0]
    copies = [
        pltpu.make_async_copy(iq_ref.at[r], out_ref.at[r], sem_big)
        for r in range(NUM_STEPS)
    ]
    for c in copies:
        c.start()
    cp_row = pltpu.make_async_copy(iq_ref.at[tt], vrow, sem_row)
    cp_qsd = pltpu.make_async_copy(qsd_ref, vqsd, sem_qsd)
    cp_row.start()
    cp_qsd.start()
    cp_row.wait()
    cp_qsd.wait()

    a = jax.lax.broadcasted_iota(jnp.int32, (_SUB, _W), 0)
    b = jax.lax.broadcasted_iota(jnp.int32, (_SUB, _W), 1)
    col = (a * _W + b).astype(jnp.uint32)
    u1 = _bits_to_unit(_threefry2x32(kd_ref[0], kd_ref[1], col))
    u2 = _bits_to_unit(_threefry2x32(kd_ref[2], kd_ref[3], col))
    p1 = jnp.clip(probs_ref[0], jnp.float32(1e-6), jnp.float32(1.0 - 1e-6))
    p2 = jnp.clip(probs_ref[1], jnp.float32(1e-6), jnp.float32(1.0 - 1e-6))
    s = (u1 > jnp.float32(1.0) - p1).astype(jnp.float32)
    brk = (u2 > jnp.float32(1.0) - p2).astype(jnp.float32)

    x = vrow[...]
    t_f = tt.astype(jnp.float32)
    end = (t_f >= vqsd[...] + jnp.float32(QUARANTINE_DAYS)).astype(jnp.float32)
    r0 = x * (jnp.float32(1.0) - end)
    r1 = r0 + (jnp.float32(1.0) - r0) * ((jnp.float32(1.0) - r0) * s)
    vout[...] = r1 * (jnp.float32(1.0) - r1 * brk)

    for c in copies:
        c.wait()
    wr = pltpu.make_async_copy(vout, out_ref.at[tt], sem_wr)
    wr.start()
    wr.wait()


@jax.jit
def kernel(is_quarantined, quarantine_start_date, quarantine_start_prob,
           quarantine_break_prob, t):
    num_steps, n = is_quarantined.shape
    key = jax.random.fold_in(jax.random.key(1), t)
    k1, k2 = jax.random.split(key)
    kd = jnp.concatenate(
        [jax.random.key_data(k1), jax.random.key_data(k2)]
    ).astype(jnp.uint32)
    probs = jnp.stack(
        [quarantine_start_prob[0], quarantine_break_prob[0]]
    ).astype(jnp.float32)
    t32 = jnp.asarray(t, jnp.int32).reshape(1)
    iq3 = is_quarantined.reshape(num_steps, _SUB, _W)
    qsd2 = quarantine_start_date.astype(jnp.float32).reshape(_SUB, _W)

    out = pl.pallas_call(
        _body,
        in_specs=[
            pl.BlockSpec(memory_space=pltpu.SMEM),
            pl.BlockSpec(memory_space=pltpu.SMEM),
            pl.BlockSpec(memory_space=pltpu.SMEM),
            pl.BlockSpec(memory_space=pl.ANY),
            pl.BlockSpec(memory_space=pl.ANY),
        ],
        out_specs=pl.BlockSpec(memory_space=pl.ANY),
        out_shape=jax.ShapeDtypeStruct((num_steps, _SUB, _W), jnp.float32),
        scratch_shapes=[
            pltpu.VMEM((_SUB, _W), jnp.float32),
            pltpu.VMEM((_SUB, _W), jnp.float32),
            pltpu.VMEM((_SUB, _W), jnp.float32),
            pltpu.SemaphoreType.DMA,
            pltpu.SemaphoreType.DMA,
            pltpu.SemaphoreType.DMA,
            pltpu.SemaphoreType.DMA,
        ],
    )(kd, probs, t32, iq3, qsd2)
    return out.reshape(num_steps, n)


# speedup vs baseline: 41.7994x; 41.7994x over previous
import functools

import jax
import jax.numpy as jnp
import numpy as np
from jax.experimental import pallas as pl
from jax.experimental.pallas import tpu as pltpu

NUM_STEPS = 64
NUM_AGENTS = 500000
QUARANTINE_DAYS = 10.0
_BC = 32768


def _threefry2x32(k0, k1, x1_in):
    ks0 = k0
    ks1 = k1
    ks2 = k0 ^ k1 ^ jnp.uint32(0x1BD11BDA)
    ks = (ks0, ks1, ks2)
    x0 = jnp.zeros_like(x1_in) + ks0
    x1 = x1_in + ks1
    rotations = ((13, 15, 26, 6), (17, 29, 16, 24))
    for i in range(5):
        for r in rotations[i % 2]:
            x0 = x0 + x1
            x1 = (x1 << jnp.uint32(r)) | (x1 >> jnp.uint32(32 - r))
            x1 = x1 ^ x0
        x0 = x0 + ks[(i + 1) % 3]
        x1 = x1 + ks[(i + 2) % 3] + jnp.uint32(i + 1)
    return x0 ^ x1


def _bits_to_unit(bits):
    f = jax.lax.bitcast_convert_type(
        (bits >> jnp.uint32(9)) | jnp.uint32(0x3F800000), jnp.float32
    ) - jnp.float32(1.0)
    minv = jnp.float32(1e-6)
    maxv = jnp.float32(1.0 - 1e-6)
    return jnp.maximum(minv, f * (maxv - minv) + minv)


def _body(kd_ref, probs_ref, t_ref, iq_ref, qsd_ref, out_ref):
    i = pl.program_id(0)
    tt = t_ref[0]
    bsub = _BC // 8
    a = jax.lax.broadcasted_iota(jnp.int32, (8, bsub), 0)
    b = jax.lax.broadcasted_iota(jnp.int32, (8, bsub), 1)
    col = (i * _BC + a * bsub + b).astype(jnp.uint32)
    bits1 = _threefry2x32(kd_ref[0], kd_ref[1], col)
    bits2 = _threefry2x32(kd_ref[2], kd_ref[3], col)
    u1 = _bits_to_unit(bits1)
    u2 = _bits_to_unit(bits2)
    p1 = jnp.clip(probs_ref[0], jnp.float32(1e-6), jnp.float32(1.0 - 1e-6))
    p2 = jnp.clip(probs_ref[1], jnp.float32(1e-6), jnp.float32(1.0 - 1e-6))
    s = (u1 > jnp.float32(1.0) - p1).astype(jnp.float32).reshape(1, _BC)
    brk = (u2 > jnp.float32(1.0) - p2).astype(jnp.float32).reshape(1, _BC)

    x = iq_ref[...]
    t_f = tt.astype(jnp.float32)
    end = (t_f >= qsd_ref[...] + jnp.float32(QUARANTINE_DAYS)).astype(jnp.float32)
    r0 = x * (jnp.float32(1.0) - end)
    r1 = r0 + (jnp.float32(1.0) - r0) * ((jnp.float32(1.0) - r0) * s)
    r2 = r1 * (jnp.float32(1.0) - r1 * brk)
    rows = jax.lax.broadcasted_iota(jnp.int32, (64, _BC), 0)
    out_ref[...] = jnp.where(rows == tt, r2, x)


@jax.jit
def kernel(is_quarantined, quarantine_start_date, quarantine_start_prob,
           quarantine_break_prob, t):
    num_steps, n = is_quarantined.shape
    key = jax.random.fold_in(jax.random.key(1), t)
    k1, k2 = jax.random.split(key)
    kd = jnp.concatenate(
        [jax.random.key_data(k1), jax.random.key_data(k2)]
    ).astype(jnp.uint32)
    probs = jnp.stack(
        [quarantine_start_prob[0], quarantine_break_prob[0]]
    ).astype(jnp.float32)
    t32 = jnp.asarray(t, jnp.int32).reshape(1)
    qsd = quarantine_start_date.astype(jnp.float32).reshape(1, n)

    grid = pl.cdiv(n, _BC)
    out = pl.pallas_call(
        _body,
        grid=(grid,),
        in_specs=[
            pl.BlockSpec(memory_space=pltpu.SMEM),
            pl.BlockSpec(memory_space=pltpu.SMEM),
            pl.BlockSpec(memory_space=pltpu.SMEM),
            pl.BlockSpec((num_steps, _BC), lambda i: (0, i)),
            pl.BlockSpec((1, _BC), lambda i: (0, i)),
        ],
        out_specs=pl.BlockSpec((num_steps, _BC), lambda i: (0, i)),
        out_shape=jax.ShapeDtypeStruct((num_steps, n), jnp.float32),
    )(kd, probs, t32, is_quarantined, qsd)
    return out
